# Initial kernel scaffold; baseline (speedup 1.0000x reference)
import functools

import jax
import jax.numpy as jnp
from jax.experimental import pallas as pl

_B = 16
_N = 1024
_EPS = 1e-5


def _cumsum_lanes(x, n):
    q = x.shape[0]
    s = 1
    while s < n:
        shifted = jnp.concatenate(
            [jnp.zeros((q, s), x.dtype), x[:, : n - s]], axis=1)
        x = x + shifted
        s *= 2
    return x


def _fps_body(xcm_ref, xrm_ref, out_ref, *, npoint, n):
    xcm = xcm_ref[0]
    x0 = xcm[0:1, :]
    x1 = xcm[1:2, :]
    x2 = xcm[2:3, :]
    lane = jax.lax.broadcasted_iota(jnp.int32, (1, n), 1)

    def step(i, carry):
        dist, far = carry
        crow = xrm_ref[0, pl.ds(far, 1), :]
        out_ref[0, pl.ds(i, 1), :] = crow
        cx = jnp.sum(crow[:, 0:1])
        cy = jnp.sum(crow[:, 1:2])
        cz = jnp.sum(crow[:, 2:3])
        d = (x0 - cx) ** 2 + (x1 - cy) ** 2 + (x2 - cz) ** 2
        dist = jnp.minimum(dist, d)
        m = jnp.max(dist)
        far = jnp.min(jnp.where(dist == m, lane, n))
        return dist, far

    jax.lax.fori_loop(
        0, npoint, step,
        (jnp.full((1, n), 1e10, jnp.float32), jnp.int32(0)))


def _fps(xyz_cm, xyz_rm, npoint, n):
    return pl.pallas_call(
        functools.partial(_fps_body, npoint=npoint, n=n),
        grid=(_B,),
        in_specs=[
            pl.BlockSpec((1, 3, n), lambda b: (b, 0, 0)),
            pl.BlockSpec((1, n, 3), lambda b: (b, 0, 0)),
        ],
        out_specs=pl.BlockSpec((1, npoint, 3), lambda b: (b, 0, 0)),
        out_shape=jax.ShapeDtypeStruct((_B, npoint, 3), jnp.float32),
    )(xyz_cm, xyz_rm)


def _select(onehot, table, found, q, k):
    g = jax.lax.dot_general(
        onehot, table, (((1,), (0,)), ((), ())),
        preferred_element_type=jnp.float32)
    c = g.shape[-1]
    g3 = g.reshape(q, k, c)
    return jnp.where(found[:, :, None], g3, g3[:, 0:1, :])


def _ballgroup_math(xcm_ref, new_ref, *, n, k, q, r2):
    cen = new_ref[0]
    xcm = xcm_ref[0]
    m = jax.lax.dot_general(
        cen, xcm, (((1,), (0,)), ((), ())),
        preferred_element_type=jnp.float32)
    d = -2.0 * m
    d = d + jnp.sum(cen * cen, axis=1, keepdims=True)
    d = d + jnp.sum(xcm * xcm, axis=0, keepdims=True)
    keep = jnp.logical_not(d > r2)
    c = _cumsum_lanes(keep.astype(jnp.float32), n)
    ranksel = jnp.where(keep, c, 0.0)
    cnt = c[:, n - 1:n]
    kcol = jax.lax.broadcasted_iota(jnp.float32, (q, k, 1), 1) + 1.0
    onehot = (ranksel[:, None, :] == kcol).astype(jnp.float32)
    onehot = onehot.reshape(q * k, n)
    ki = jax.lax.broadcasted_iota(jnp.float32, (q, k), 1)
    found = ki < cnt
    return cen, onehot, found


def _bg_body(xcm_ref, xrm_ref, new_ref, ox_ref, *, n, k, q, r2):
    cen, onehot, found = _ballgroup_math(xcm_ref, new_ref, n=n, k=k, q=q, r2=r2)
    sel = _select(onehot, xrm_ref[0], found, q, k)
    ox_ref[0] = (sel - cen[:, None, :]).reshape(q * k, 3)


def _bg_feats_body(xcm_ref, xrm_ref, f_ref, new_ref, ox_ref, of_ref,
                   *, n, k, q, r2):
    cen, onehot, found = _ballgroup_math(xcm_ref, new_ref, n=n, k=k, q=q, r2=r2)
    sel = _select(onehot, xrm_ref[0], found, q, k)
    ox_ref[0] = (sel - cen[:, None, :]).reshape(q * k, 3)
    fsel = _select(onehot, f_ref[0], found, q, k)
    of_ref[0] = fsel.reshape(q * k, f_ref.shape[-1])


def _ballgroup(xyz_cm, xyz_rm, new_xyz, s, k, n, radius, q, feats=None):
    r2 = radius * radius
    grid = (_B, s // q)
    in_specs = [
        pl.BlockSpec((1, 3, n), lambda b, c: (b, 0, 0)),
        pl.BlockSpec((1, n, 3), lambda b, c: (b, 0, 0)),
    ]
    args = [xyz_cm, xyz_rm]
    if feats is None:
        body = functools.partial(_bg_body, n=n, k=k, q=q, r2=r2)
        out_specs = pl.BlockSpec((1, q * k, 3), lambda b, c: (b, c, 0))
        out_shape = jax.ShapeDtypeStruct((_B, s * k, 3), jnp.float32)
    else:
        cf = feats.shape[-1]
        body = functools.partial(_bg_feats_body, n=n, k=k, q=q, r2=r2)
        in_specs.append(pl.BlockSpec((1, n, cf), lambda b, c: (b, 0, 0)))
        args.append(feats)
        out_specs = [
            pl.BlockSpec((1, q * k, 3), lambda b, c: (b, c, 0)),
            pl.BlockSpec((1, q * k, cf), lambda b, c: (b, c, 0)),
        ]
        out_shape = [
            jax.ShapeDtypeStruct((_B, s * k, 3), jnp.float32),
            jax.ShapeDtypeStruct((_B, s * k, cf), jnp.float32),
        ]
    in_specs.append(pl.BlockSpec((1, q, 3), lambda b, c: (b, c, 0)))
    args.append(new_xyz)
    return pl.pallas_call(
        body, grid=grid, in_specs=in_specs, out_specs=out_specs,
        out_shape=out_shape)(*args)


def _stats_body(x_ref, w_ref, b_ref, out_ref):
    y = jnp.dot(x_ref[...], w_ref[...],
                preferred_element_type=jnp.float32) + b_ref[...]
    p = jnp.concatenate(
        [jnp.sum(y, 0, keepdims=True), jnp.sum(y * y, 0, keepdims=True)], 0)

    @pl.when(pl.program_id(0) == 0)
    def _():
        out_ref[...] = p

    @pl.when(pl.program_id(0) != 0)
    def _():
        out_ref[...] = out_ref[...] + p


def _stats2_body(xa_ref, xb_ref, wa_ref, wb_ref, b_ref, out_ref):
    y = (jnp.dot(xa_ref[...], wa_ref[...], preferred_element_type=jnp.float32)
         + jnp.dot(xb_ref[...], wb_ref[...], preferred_element_type=jnp.float32)
         + b_ref[...])
    p = jnp.concatenate(
        [jnp.sum(y, 0, keepdims=True), jnp.sum(y * y, 0, keepdims=True)], 0)

    @pl.when(pl.program_id(0) == 0)
    def _():
        out_ref[...] = p

    @pl.when(pl.program_id(0) != 0)
    def _():
        out_ref[...] = out_ref[...] + p


def _bn_from_stats(y, s_ref, g_ref, be_ref, rows):
    mu = s_ref[0:1, :] / rows
    var = s_ref[1:2, :] / rows - mu * mu
    z = g_ref[...] * (y - mu) / jnp.sqrt(var + _EPS) + be_ref[...]
    return jnp.maximum(z, 0.0)


def _apply_body(x_ref, w_ref, b_ref, g_ref, be_ref, s_ref, o_ref,
                *, rows, pool):
    y = jnp.dot(x_ref[...], w_ref[...],
                preferred_element_type=jnp.float32) + b_ref[...]
    z = _bn_from_stats(y, s_ref, g_ref, be_ref, rows)
    if pool is None:
        o_ref[...] = z
    else:
        br, c = z.shape
        o_ref[...] = z.reshape(br // pool, pool, c).max(axis=1)


def _apply2_body(xa_ref, xb_ref, wa_ref, wb_ref, b_ref, g_ref, be_ref,
                 s_ref, o_ref, *, rows):
    y = (jnp.dot(xa_ref[...], wa_ref[...], preferred_element_type=jnp.float32)
         + jnp.dot(xb_ref[...], wb_ref[...], preferred_element_type=jnp.float32)
         + b_ref[...])
    o_ref[...] = _bn_from_stats(y, s_ref, g_ref, be_ref, rows)


def _row2(v):
    return v.reshape(1, -1)


def _full(shape):
    nd = len(shape)
    return pl.BlockSpec(shape, lambda i: (0,) * nd)


def _dense_bn_relu(x, w, b, g, be, br, pool=None):
    rows, cin = x.shape
    cout = w.shape[1]
    grid = (rows // br,)
    b2, g2, be2 = _row2(b), _row2(g), _row2(be)
    stats = pl.pallas_call(
        _stats_body, grid=grid,
        in_specs=[pl.BlockSpec((br, cin), lambda i: (i, 0)),
                  _full((cin, cout)), _full((1, cout))],
        out_specs=_full((2, cout)),
        out_shape=jax.ShapeDtypeStruct((2, cout), jnp.float32),
    )(x, w, b2)
    if pool is None:
        out_shape = jax.ShapeDtypeStruct((rows, cout), jnp.float32)
        out_specs = pl.BlockSpec((br, cout), lambda i: (i, 0))
    else:
        out_shape = jax.ShapeDtypeStruct((rows // pool, cout), jnp.float32)
        out_specs = pl.BlockSpec((br // pool, cout), lambda i: (i, 0))
    return pl.pallas_call(
        functools.partial(_apply_body, rows=float(rows), pool=pool),
        grid=grid,
        in_specs=[pl.BlockSpec((br, cin), lambda i: (i, 0)),
                  _full((cin, cout)), _full((1, cout)), _full((1, cout)),
                  _full((1, cout)), _full((2, cout))],
        out_specs=out_specs, out_shape=out_shape,
    )(x, w, b2, g2, be2, stats)


def _dense_bn_relu2(xa, xb, wa, wb, b, g, be, br):
    rows = xa.shape[0]
    ca, cb = xa.shape[1], xb.shape[1]
    cout = wa.shape[1]
    grid = (rows // br,)
    b2, g2, be2 = _row2(b), _row2(g), _row2(be)
    stats = pl.pallas_call(
        _stats2_body, grid=grid,
        in_specs=[pl.BlockSpec((br, ca), lambda i: (i, 0)),
                  pl.BlockSpec((br, cb), lambda i: (i, 0)),
                  _full((ca, cout)), _full((cb, cout)), _full((1, cout))],
        out_specs=_full((2, cout)),
        out_shape=jax.ShapeDtypeStruct((2, cout), jnp.float32),
    )(xa, xb, wa, wb, b2)
    return pl.pallas_call(
        functools.partial(_apply2_body, rows=float(rows)),
        grid=grid,
        in_specs=[pl.BlockSpec((br, ca), lambda i: (i, 0)),
                  pl.BlockSpec((br, cb), lambda i: (i, 0)),
                  _full((ca, cout)), _full((cb, cout)), _full((1, cout)),
                  _full((1, cout)), _full((1, cout)), _full((2, cout))],
        out_specs=pl.BlockSpec((br, cout), lambda i: (i, 0)),
        out_shape=jax.ShapeDtypeStruct((rows, cout), jnp.float32),
    )(xa, xb, wa, wb, b2, g2, be2, stats)


def _bn_local_relu(y, g, be):
    mu = jnp.mean(y, axis=0, keepdims=True)
    var = jnp.mean((y - mu) ** 2, axis=0, keepdims=True)
    return jnp.maximum(g * (y - mu) / jnp.sqrt(var + _EPS) + be, 0.0)


def _sa3_body(xyz_ref, f_ref, wx_ref, wf_ref, b1_ref, g1_ref, be1_ref,
              w2_ref, b2_ref, g2_ref, be2_ref,
              w3_ref, b3_ref, g3_ref, be3_ref, out_ref, *, s, cf):
    xyz = xyz_ref[...]
    rel = (xyz - xyz[:, 0:1, :]).reshape(_B * s, 3)
    f = f_ref[...].reshape(_B * s, cf)
    y = (jnp.dot(rel, wx_ref[...], preferred_element_type=jnp.float32)
         + jnp.dot(f, wf_ref[...], preferred_element_type=jnp.float32)
         + b1_ref[...])
    y = _bn_local_relu(y, g1_ref[...], be1_ref[...])
    y = jnp.dot(y, w2_ref[...], preferred_element_type=jnp.float32) + b2_ref[...]
    y = _bn_local_relu(y, g2_ref[...], be2_ref[...])
    y = jnp.dot(y, w3_ref[...], preferred_element_type=jnp.float32) + b3_ref[...]
    y = _bn_local_relu(y, g3_ref[...], be3_ref[...])
    out_ref[...] = y.reshape(_B, s, y.shape[-1]).max(axis=1)


def _fc_body(x_ref, w1_ref, b1_ref, g1_ref, be1_ref,
             w2_ref, b2_ref, g2_ref, be2_ref, w3_ref, b3_ref, out_ref):
    h = jnp.dot(x_ref[...], w1_ref[...],
                preferred_element_type=jnp.float32) + b1_ref[...]
    h = _bn_local_relu(h, g1_ref[...], be1_ref[...])
    h = jnp.dot(h, w2_ref[...], preferred_element_type=jnp.float32) + b2_ref[...]
    h = _bn_local_relu(h, g2_ref[...], be2_ref[...])
    out_ref[...] = jnp.dot(h, w3_ref[...],
                           preferred_element_type=jnp.float32) + b3_ref[...]


def kernel(inputs, sa1_params, sa2_params, sa3_params, fc_params):
    xyz_cm = inputs
    xyz_rm = jnp.swapaxes(inputs, 1, 2)

    s1, k1 = 512, 32
    new1 = _fps(xyz_cm, xyz_rm, s1, _N)
    g1 = _ballgroup(xyz_cm, xyz_rm, new1, s1, k1, _N, 0.2, q=64)
    x = g1.reshape(_B * s1 * k1, 3)
    (w1, b1, ga1, be1), (w2, b2, ga2, be2), (w3, b3, ga3, be3) = sa1_params
    x = _dense_bn_relu(x, w1, b1, ga1, be1, br=4096)
    x = _dense_bn_relu(x, w2, b2, ga2, be2, br=4096)
    l1_f = _dense_bn_relu(x, w3, b3, ga3, be3, br=4096, pool=k1)
    l1_f = l1_f.reshape(_B, s1, 128)

    s2, k2 = 128, 64
    l1_cm = jnp.swapaxes(new1, 1, 2)
    new2 = _fps(l1_cm, new1, s2, s1)
    g2x, g2f = _ballgroup(l1_cm, new1, new2, s2, k2, s1, 0.4, q=64,
                          feats=l1_f)
    xa = g2x.reshape(_B * s2 * k2, 3)
    xb = g2f.reshape(_B * s2 * k2, 128)
    (w1, b1, ga1, be1), (w2, b2, ga2, be2), (w3, b3, ga3, be3) = sa2_params
    x = _dense_bn_relu2(xa, xb, w1[0:3], w1[3:131], b1, ga1, be1, br=4096)
    x = _dense_bn_relu(x, w2, b2, ga2, be2, br=4096)
    l2_f = _dense_bn_relu(x, w3, b3, ga3, be3, br=4096, pool=k2)
    l2_f = l2_f.reshape(_B, s2, 256)

    (w1, b1, ga1, be1), (w2, b2, ga2, be2), (w3, b3, ga3, be3) = sa3_params
    l3_f = pl.pallas_call(
        functools.partial(_sa3_body, s=s2, cf=256),
        out_shape=jax.ShapeDtypeStruct((_B, 1024), jnp.float32),
    )(new2, l2_f, w1[0:3], w1[3:259], _row2(b1), _row2(ga1), _row2(be1),
      w2, _row2(b2), _row2(ga2), _row2(be2),
      w3, _row2(b3), _row2(ga3), _row2(be3))

    (fw1, fb1, fg1, fbe1), (fw2, fb2, fg2, fbe2), (fw3, fb3) = fc_params
    return pl.pallas_call(
        _fc_body,
        out_shape=jax.ShapeDtypeStruct((_B, 40), jnp.float32),
    )(l3_f, fw1, _row2(fb1), _row2(fg1), _row2(fbe1),
      fw2, _row2(fb2), _row2(fg2), _row2(fbe2), fw3, _row2(fb3))


# speedup vs baseline: 2.1201x; 2.1201x over previous
import functools

import jax
import jax.numpy as jnp
from jax.experimental import pallas as pl

_B = 16
_N = 1024
_EPS = 1e-5
_PREC = jax.lax.Precision.DEFAULT


def _cumsum_lanes(x, n):
    q = x.shape[0]
    s = 1
    while s < n:
        shifted = jnp.concatenate(
            [jnp.zeros((q, s), x.dtype), x[:, : n - s]], axis=1)
        x = x + shifted
        s *= 2
    return x


def _fps_body(xcm_ref, xrm_ref, out_ref, *, npoint, n):
    xcm = xcm_ref[0]
    x0 = xcm[0:1, :]
    x1 = xcm[1:2, :]
    x2 = xcm[2:3, :]
    lane = jax.lax.broadcasted_iota(jnp.int32, (1, n), 1)

    def step(i, carry):
        dist, far = carry
        crow = xrm_ref[0, pl.ds(far, 1), :]
        out_ref[0, pl.ds(i, 1), :] = crow
        cx = jnp.sum(crow[:, 0:1])
        cy = jnp.sum(crow[:, 1:2])
        cz = jnp.sum(crow[:, 2:3])
        d = (x0 - cx) ** 2 + (x1 - cy) ** 2 + (x2 - cz) ** 2
        dist = jnp.minimum(dist, d)
        m = jnp.max(dist)
        far = jnp.min(jnp.where(dist == m, lane, n))
        return dist, far

    jax.lax.fori_loop(
        0, npoint, step,
        (jnp.full((1, n), 1e10, jnp.float32), jnp.int32(0)))


def _fps(xyz_cm, xyz_rm, npoint, n):
    return pl.pallas_call(
        functools.partial(_fps_body, npoint=npoint, n=n),
        grid=(_B,),
        in_specs=[
            pl.BlockSpec((1, 3, n), lambda b: (b, 0, 0)),
            pl.BlockSpec((1, n, 3), lambda b: (b, 0, 0)),
        ],
        out_specs=pl.BlockSpec((1, npoint, 3), lambda b: (b, 0, 0)),
        out_shape=jax.ShapeDtypeStruct((_B, npoint, 3), jnp.float32),
    )(xyz_cm, xyz_rm)


def _select(onehot, table, notfound, q, k):
    g = jax.lax.dot_general(
        onehot, table, (((1,), (0,)), ((), ())),
        preferred_element_type=jnp.float32,
        precision=jax.lax.Precision.HIGHEST)
    c = g.shape[-1]
    g3 = g.reshape(q, k, c)
    return g3 + notfound * g3[:, 0:1, :]


def _ballgroup_math(xcm_ref, new_ref, *, n, k, q, r2):
    cen = new_ref[0]
    xcm = xcm_ref[0]
    m = jax.lax.dot_general(
        cen, xcm, (((1,), (0,)), ((), ())),
        preferred_element_type=jnp.float32)
    d = -2.0 * m
    d = d + jnp.sum(cen * cen, axis=1, keepdims=True)
    d = d + jnp.sum(xcm * xcm, axis=0, keepdims=True)
    keep = jnp.logical_not(d > r2)
    c = _cumsum_lanes(keep.astype(jnp.float32), n)
    ranksel = jnp.where(keep, c, 0.0)
    rank3 = ranksel[:, None, :]
    kcol = jax.lax.broadcasted_iota(
        jnp.int32, (q, k, 1), 1).astype(jnp.float32) + 1.0
    onehot = (rank3 == kcol).astype(jnp.float32)
    onehot = onehot.reshape(q * k, n)
    cnt3 = jnp.max(rank3, axis=2, keepdims=True)
    notfound = (kcol > cnt3).astype(jnp.float32)
    return cen, onehot, notfound


def _bg_body(xcm_ref, xrm_ref, new_ref, ox_ref, *, n, k, q, r2):
    cen, onehot, nf = _ballgroup_math(xcm_ref, new_ref, n=n, k=k, q=q, r2=r2)
    sel = _select(onehot, xrm_ref[0], nf, q, k)
    ox_ref[0] = (sel - cen[:, None, :]).reshape(q * k, 3)


def _bg_feats_body(xcm_ref, xrm_ref, f_ref, new_ref, ox_ref, of_ref,
                   *, n, k, q, r2):
    cen, onehot, nf = _ballgroup_math(xcm_ref, new_ref, n=n, k=k, q=q, r2=r2)
    sel = _select(onehot, xrm_ref[0], nf, q, k)
    ox_ref[0] = (sel - cen[:, None, :]).reshape(q * k, 3)
    fsel = _select(onehot, f_ref[0], nf, q, k)
    of_ref[0] = fsel.reshape(q * k, f_ref.shape[-1])


def _ballgroup(xyz_cm, xyz_rm, new_xyz, s, k, n, radius, q, feats=None):
    r2 = radius * radius
    grid = (_B, s // q)
    in_specs = [
        pl.BlockSpec((1, 3, n), lambda b, c: (b, 0, 0)),
        pl.BlockSpec((1, n, 3), lambda b, c: (b, 0, 0)),
    ]
    args = [xyz_cm, xyz_rm]
    if feats is None:
        body = functools.partial(_bg_body, n=n, k=k, q=q, r2=r2)
        out_specs = pl.BlockSpec((1, q * k, 3), lambda b, c: (b, c, 0))
        out_shape = jax.ShapeDtypeStruct((_B, s * k, 3), jnp.float32)
    else:
        cf = feats.shape[-1]
        body = functools.partial(_bg_feats_body, n=n, k=k, q=q, r2=r2)
        in_specs.append(pl.BlockSpec((1, n, cf), lambda b, c: (b, 0, 0)))
        args.append(feats)
        out_specs = [
            pl.BlockSpec((1, q * k, 3), lambda b, c: (b, c, 0)),
            pl.BlockSpec((1, q * k, cf), lambda b, c: (b, c, 0)),
        ]
        out_shape = [
            jax.ShapeDtypeStruct((_B, s * k, 3), jnp.float32),
            jax.ShapeDtypeStruct((_B, s * k, cf), jnp.float32),
        ]
    in_specs.append(pl.BlockSpec((1, q, 3), lambda b, c: (b, c, 0)))
    args.append(new_xyz)
    return pl.pallas_call(
        body, grid=grid, in_specs=in_specs, out_specs=out_specs,
        out_shape=out_shape)(*args)


def _accum(out_ref, p):
    @pl.when(pl.program_id(0) == 0)
    def _():
        out_ref[...] = p

    @pl.when(pl.program_id(0) != 0)
    def _():
        out_ref[...] = out_ref[...] + p


def _sum_body(x_ref, w_ref, b_ref, out_ref):
    y = jnp.dot(x_ref[...], w_ref[...],
                preferred_element_type=jnp.float32, precision=_PREC) + b_ref[...]
    _accum(out_ref, jnp.sum(y, 0, keepdims=True))


def _sq_body(x_ref, w_ref, b_ref, s_ref, out_ref, *, rows):
    y = jnp.dot(x_ref[...], w_ref[...],
                preferred_element_type=jnp.float32, precision=_PREC) + b_ref[...]
    dev = y - s_ref[...] / rows
    _accum(out_ref, jnp.sum(dev * dev, 0, keepdims=True))


def _sum2_body(xa_ref, xb_ref, wa_ref, wb_ref, b_ref, out_ref):
    y = (jnp.dot(xa_ref[...], wa_ref[...], preferred_element_type=jnp.float32, precision=_PREC)
         + jnp.dot(xb_ref[...], wb_ref[...], preferred_element_type=jnp.float32, precision=_PREC)
         + b_ref[...])
    _accum(out_ref, jnp.sum(y, 0, keepdims=True))


def _sq2_body(xa_ref, xb_ref, wa_ref, wb_ref, b_ref, s_ref, out_ref, *, rows):
    y = (jnp.dot(xa_ref[...], wa_ref[...], preferred_element_type=jnp.float32, precision=_PREC)
         + jnp.dot(xb_ref[...], wb_ref[...], preferred_element_type=jnp.float32, precision=_PREC)
         + b_ref[...])
    dev = y - s_ref[...] / rows
    _accum(out_ref, jnp.sum(dev * dev, 0, keepdims=True))


def _bn_from_stats(y, s_ref, q_ref, g_ref, be_ref, rows):
    mu = s_ref[...] / rows
    var = q_ref[...] / rows
    z = g_ref[...] * (y - mu) / jnp.sqrt(var + _EPS) + be_ref[...]
    return jnp.maximum(z, 0.0)


def _apply_body(x_ref, w_ref, b_ref, g_ref, be_ref, s_ref, q_ref, o_ref,
                *, rows, pool):
    y = jnp.dot(x_ref[...], w_ref[...],
                preferred_element_type=jnp.float32, precision=_PREC) + b_ref[...]
    z = _bn_from_stats(y, s_ref, q_ref, g_ref, be_ref, rows)
    if pool is None:
        o_ref[...] = z
    else:
        br, c = z.shape
        o_ref[...] = z.reshape(br // pool, pool, c).max(axis=1)


def _apply2_body(xa_ref, xb_ref, wa_ref, wb_ref, b_ref, g_ref, be_ref,
                 s_ref, q_ref, o_ref, *, rows):
    y = (jnp.dot(xa_ref[...], wa_ref[...], preferred_element_type=jnp.float32, precision=_PREC)
         + jnp.dot(xb_ref[...], wb_ref[...], preferred_element_type=jnp.float32, precision=_PREC)
         + b_ref[...])
    o_ref[...] = _bn_from_stats(y, s_ref, q_ref, g_ref, be_ref, rows)


def _row2(v):
    return v.reshape(1, -1)


def _full(shape):
    nd = len(shape)
    return pl.BlockSpec(shape, lambda i: (0,) * nd)


def _dense_bn_relu(x, w, b, g, be, br, pool=None):
    rows, cin = x.shape
    cout = w.shape[1]
    grid = (rows // br,)
    b2, g2, be2 = _row2(b), _row2(g), _row2(be)
    xspec = pl.BlockSpec((br, cin), lambda i: (i, 0))
    ssum = pl.pallas_call(
        _sum_body, grid=grid,
        in_specs=[xspec, _full((cin, cout)), _full((1, cout))],
        out_specs=_full((1, cout)),
        out_shape=jax.ShapeDtypeStruct((1, cout), jnp.float32),
    )(x, w, b2)
    ssq = pl.pallas_call(
        functools.partial(_sq_body, rows=float(rows)), grid=grid,
        in_specs=[xspec, _full((cin, cout)), _full((1, cout)),
                  _full((1, cout))],
        out_specs=_full((1, cout)),
        out_shape=jax.ShapeDtypeStruct((1, cout), jnp.float32),
    )(x, w, b2, ssum)
    if pool is None:
        out_shape = jax.ShapeDtypeStruct((rows, cout), jnp.float32)
        out_specs = pl.BlockSpec((br, cout), lambda i: (i, 0))
    else:
        out_shape = jax.ShapeDtypeStruct((rows // pool, cout), jnp.float32)
        out_specs = pl.BlockSpec((br // pool, cout), lambda i: (i, 0))
    return pl.pallas_call(
        functools.partial(_apply_body, rows=float(rows), pool=pool),
        grid=grid,
        in_specs=[xspec, _full((cin, cout)), _full((1, cout)),
                  _full((1, cout)), _full((1, cout)), _full((1, cout)),
                  _full((1, cout))],
        out_specs=out_specs, out_shape=out_shape,
    )(x, w, b2, g2, be2, ssum, ssq)


def _dense_bn_relu2(xa, xb, wa, wb, b, g, be, br):
    rows = xa.shape[0]
    ca, cb = xa.shape[1], xb.shape[1]
    cout = wa.shape[1]
    grid = (rows // br,)
    b2, g2, be2 = _row2(b), _row2(g), _row2(be)
    aspec = pl.BlockSpec((br, ca), lambda i: (i, 0))
    bspec = pl.BlockSpec((br, cb), lambda i: (i, 0))
    ssum = pl.pallas_call(
        _sum2_body, grid=grid,
        in_specs=[aspec, bspec,
                  _full((ca, cout)), _full((cb, cout)), _full((1, cout))],
        out_specs=_full((1, cout)),
        out_shape=jax.ShapeDtypeStruct((1, cout), jnp.float32),
    )(xa, xb, wa, wb, b2)
    ssq = pl.pallas_call(
        functools.partial(_sq2_body, rows=float(rows)), grid=grid,
        in_specs=[aspec, bspec,
                  _full((ca, cout)), _full((cb, cout)), _full((1, cout)),
                  _full((1, cout))],
        out_specs=_full((1, cout)),
        out_shape=jax.ShapeDtypeStruct((1, cout), jnp.float32),
    )(xa, xb, wa, wb, b2, ssum)
    return pl.pallas_call(
        functools.partial(_apply2_body, rows=float(rows)),
        grid=grid,
        in_specs=[aspec, bspec,
                  _full((ca, cout)), _full((cb, cout)), _full((1, cout)),
                  _full((1, cout)), _full((1, cout)), _full((1, cout)),
                  _full((1, cout))],
        out_specs=pl.BlockSpec((br, cout), lambda i: (i, 0)),
        out_shape=jax.ShapeDtypeStruct((rows, cout), jnp.float32),
    )(xa, xb, wa, wb, b2, g2, be2, ssum, ssq)


def _bn_local_relu(y, g, be):
    mu = jnp.mean(y, axis=0, keepdims=True)
    var = jnp.mean((y - mu) ** 2, axis=0, keepdims=True)
    return jnp.maximum(g * (y - mu) / jnp.sqrt(var + _EPS) + be, 0.0)


def _sa3_body(xyz_ref, f_ref, wx_ref, wf_ref, b1_ref, g1_ref, be1_ref,
              w2_ref, b2_ref, g2_ref, be2_ref,
              w3_ref, b3_ref, g3_ref, be3_ref, out_ref, *, s, cf):
    xyz = xyz_ref[...]
    rel = (xyz - xyz[:, 0:1, :]).reshape(_B * s, 3)
    f = f_ref[...].reshape(_B * s, cf)
    y = (jnp.dot(rel, wx_ref[...], preferred_element_type=jnp.float32, precision=_PREC)
         + jnp.dot(f, wf_ref[...], preferred_element_type=jnp.float32, precision=_PREC)
         + b1_ref[...])
    y = _bn_local_relu(y, g1_ref[...], be1_ref[...])
    y = jnp.dot(y, w2_ref[...], preferred_element_type=jnp.float32, precision=_PREC) + b2_ref[...]
    y = _bn_local_relu(y, g2_ref[...], be2_ref[...])
    y = jnp.dot(y, w3_ref[...], preferred_element_type=jnp.float32, precision=_PREC) + b3_ref[...]
    y = _bn_local_relu(y, g3_ref[...], be3_ref[...])
    out_ref[...] = y.reshape(_B, s, y.shape[-1]).max(axis=1)


def _fc_body(x_ref, w1_ref, b1_ref, g1_ref, be1_ref,
             w2_ref, b2_ref, g2_ref, be2_ref, w3_ref, b3_ref, out_ref):
    h = jnp.dot(x_ref[...], w1_ref[...],
                preferred_element_type=jnp.float32, precision=_PREC) + b1_ref[...]
    h = _bn_local_relu(h, g1_ref[...], be1_ref[...])
    h = jnp.dot(h, w2_ref[...], preferred_element_type=jnp.float32, precision=_PREC) + b2_ref[...]
    h = _bn_local_relu(h, g2_ref[...], be2_ref[...])
    out_ref[...] = jnp.dot(h, w3_ref[...],
                           preferred_element_type=jnp.float32, precision=_PREC) + b3_ref[...]


def kernel(inputs, sa1_params, sa2_params, sa3_params, fc_params):
    xyz_cm = inputs
    xyz_rm = jnp.swapaxes(inputs, 1, 2)

    s1, k1 = 512, 32
    new1 = _fps(xyz_cm, xyz_rm, s1, _N)
    g1 = _ballgroup(xyz_cm, xyz_rm, new1, s1, k1, _N, 0.2, q=64)
    x = g1.reshape(_B * s1 * k1, 3)
    (w1, b1, ga1, be1), (w2, b2, ga2, be2), (w3, b3, ga3, be3) = sa1_params
    x = _dense_bn_relu(x, w1, b1, ga1, be1, br=4096)
    x = _dense_bn_relu(x, w2, b2, ga2, be2, br=4096)
    l1_f = _dense_bn_relu(x, w3, b3, ga3, be3, br=4096, pool=k1)
    l1_f = l1_f.reshape(_B, s1, 128)

    s2, k2 = 128, 64
    l1_cm = jnp.swapaxes(new1, 1, 2)
    new2 = _fps(l1_cm, new1, s2, s1)
    g2x, g2f = _ballgroup(l1_cm, new1, new2, s2, k2, s1, 0.4, q=64,
                          feats=l1_f)
    xa = g2x.reshape(_B * s2 * k2, 3)
    xb = g2f.reshape(_B * s2 * k2, 128)
    (w1, b1, ga1, be1), (w2, b2, ga2, be2), (w3, b3, ga3, be3) = sa2_params
    x = _dense_bn_relu2(xa, xb, w1[0:3], w1[3:131], b1, ga1, be1, br=4096)
    x = _dense_bn_relu(x, w2, b2, ga2, be2, br=4096)
    l2_f = _dense_bn_relu(x, w3, b3, ga3, be3, br=4096, pool=k2)
    l2_f = l2_f.reshape(_B, s2, 256)

    (w1, b1, ga1, be1), (w2, b2, ga2, be2), (w3, b3, ga3, be3) = sa3_params
    l3_f = pl.pallas_call(
        functools.partial(_sa3_body, s=s2, cf=256),
        out_shape=jax.ShapeDtypeStruct((_B, 1024), jnp.float32),
    )(new2, l2_f, w1[0:3], w1[3:259], _row2(b1), _row2(ga1), _row2(be1),
      w2, _row2(b2), _row2(ga2), _row2(be2),
      w3, _row2(b3), _row2(ga3), _row2(be3))

    (fw1, fb1, fg1, fbe1), (fw2, fb2, fg2, fbe2), (fw3, fb3) = fc_params
    return pl.pallas_call(
        _fc_body,
        out_shape=jax.ShapeDtypeStruct((_B, 40), jnp.float32),
    )(l3_f, fw1, _row2(fb1), _row2(fg1), _row2(fbe1),
      fw2, _row2(fb2), _row2(fg2), _row2(fbe2), fw3, _row2(fb3))
